# baseline (speedup 1.0000x reference)
import functools

import jax
import jax.numpy as jnp
from jax import lax
from jax.experimental import pallas as pl
from jax.experimental.pallas import tpu as pltpu
from jax.experimental.pallas import tpu_sc as plsc

VOCAB = 1000
DIM = 64
BATCH = 4096
SEQ_LEN = 200
TOTAL = BATCH * SEQ_LEN

NUM_CORES = 2
NUM_SUBCORES = 16
NUM_WORKERS = NUM_CORES * NUM_SUBCORES

IDX_PER_ROW = 128
ROWS_PER_STEP = 4
CHUNK = ROWS_PER_STEP * IDX_PER_ROW
PER_WORKER = TOTAL // NUM_WORKERS
IDX_ROWS_PER_WORKER = PER_WORKER // IDX_PER_ROW
STEPS = PER_WORKER // CHUNK


def _emb_body(table_hbm, x_hbm, out_hbm, idx_v, rows_v, sem_g, sem_o):
    c = lax.axis_index("c")
    s = lax.axis_index("s")
    wid = s * NUM_CORES + c
    base_row = wid * IDX_ROWS_PER_WORKER

    pltpu.sync_copy(x_hbm.at[pl.ds(base_row, IDX_ROWS_PER_WORKER)], idx_v)

    def fire_gather(ichunk, b):
        for j in range(ROWS_PER_STEP):
            pltpu.async_copy(
                table_hbm.at[idx_v.at[ichunk * ROWS_PER_STEP + j]],
                rows_v.at[b].at[pl.ds(j * IDX_PER_ROW, IDX_PER_ROW)],
                sem_g.at[b],
            )

    def wait_gather(b):
        pltpu.make_async_copy(
            table_hbm.at[pl.ds(0, CHUNK)], rows_v.at[b], sem_g.at[b]
        ).wait()

    def fire_out(ichunk, b):
        pltpu.async_copy(
            rows_v.at[b],
            out_hbm.at[pl.ds((base_row + ichunk * ROWS_PER_STEP) * IDX_PER_ROW, CHUNK)],
            sem_o.at[b],
        )

    def wait_out(b):
        pltpu.make_async_copy(
            rows_v.at[b], out_hbm.at[pl.ds(0, CHUNK)], sem_o.at[b]
        ).wait()

    fire_gather(0, 0)
    fire_gather(1, 1)
    wait_gather(0)
    fire_out(0, 0)

    def outer(g, carry):
        for off in range(2):
            i = 2 * g + 1 + off
            b = (1 + off) % 2
            o = off % 2
            wait_out(o)
            fire_gather(i + 1, o)
            wait_gather(b)
            fire_out(i, b)
        return carry

    lax.fori_loop(0, (STEPS - 2) // 2, outer, 0)

    wait_gather(1)
    wait_out(0)
    fire_out(STEPS - 1, 1)
    wait_out(1)


@functools.partial(
    pl.kernel,
    mesh=plsc.VectorSubcoreMesh(core_axis_name="c", subcore_axis_name="s"),
    out_type=jax.ShapeDtypeStruct((TOTAL, DIM), jnp.float32),
    scratch_types=[
        pltpu.VMEM((IDX_ROWS_PER_WORKER, IDX_PER_ROW), jnp.int32),
        pltpu.VMEM((2, CHUNK, DIM), jnp.float32),
        pltpu.SemaphoreType.DMA((2,)),
        pltpu.SemaphoreType.DMA((2,)),
    ],
    compiler_params=pltpu.CompilerParams(use_tc_tiling_on_sc=False),
)
def _emb_call(table_hbm, x_hbm, out_hbm, idx_v, rows_v, sem_g, sem_o):
    _emb_body(table_hbm, x_hbm, out_hbm, idx_v, rows_v, sem_g, sem_o)


def kernel(x, table):
    xf = x.reshape(TOTAL // IDX_PER_ROW, IDX_PER_ROW).astype(jnp.int32)
    out = _emb_call(table, xf)
    return out.reshape(BATCH, SEQ_LEN, DIM)
